# baseline (speedup 1.0000x reference)
import functools

import jax
import jax.numpy as jnp
from jax import lax
from jax.experimental import pallas as pl
from jax.experimental.pallas import tpu as pltpu
from jax.experimental.pallas import tpu_sc as plsc

N = 10000
E = 320000
D = 128
L = 5
G = 64

NC = 2
NS = 16
CHUNK = 128
NCHUNK = 79
EDGES_PER_TILE = NCHUNK * CHUNK
REAL_PER_TILE = E // (NC * NS)
ROWS_PER_TILE = 640
N_PAD = NS * ROWS_PER_TILE


def _segment_sum_sc(h, src2d, dst2d):
    mesh = plsc.VectorSubcoreMesh(core_axis_name="c", subcore_axis_name="s")

    @functools.partial(
        pl.kernel,
        out_type=jax.ShapeDtypeStruct((NC, N_PAD, D), jnp.float32),
        mesh=mesh,
        scratch_types=[
            pltpu.VMEM((EDGES_PER_TILE,), jnp.int32),
            pltpu.VMEM((8, CHUNK), jnp.int32),
            pltpu.VMEM((8, CHUNK), jnp.int32),
            pltpu.VMEM((2, CHUNK, D), jnp.float32),
            pltpu.VMEM_SHARED((N_PAD, D), jnp.float32),
            pltpu.SemaphoreType.DMA,
            pltpu.SemaphoreType.DMA,
            pltpu.SemaphoreType.DMA,
            pltpu.SemaphoreType.DMA,
        ],
    )
    def k(h_hbm, src_hbm, dst_hbm, out_hbm, sidx, didxa, didxb, rows, acc,
          gsem0, gsem1, dsem0, dsem1):
        cid = lax.axis_index("c")
        sid = lax.axis_index("s")
        tid = cid * NS + sid

        zeros16 = jnp.zeros((16,), jnp.float32)

        pltpu.async_copy(
            src_hbm.at[pl.ds(tid * EDGES_PER_TILE, EDGES_PER_TILE)], sidx,
            gsem0)

        def zrow(r, carry):
            for j in range(D // 16):
                rows[0, r, pl.ds(j * 16, 16)] = zeros16
            return carry

        lax.fori_loop(0, CHUNK, zrow, 0)
        for t in range(ROWS_PER_TILE // CHUNK):
            pltpu.sync_copy(
                rows.at[0],
                acc.at[pl.ds(sid * ROWS_PER_TILE + t * CHUNK, CHUNK)])
        _rem = ROWS_PER_TILE % CHUNK
        if _rem:
            pltpu.sync_copy(
                rows.at[0, pl.ds(0, _rem)],
                acc.at[pl.ds(sid * ROWS_PER_TILE
                             + (ROWS_PER_TILE // CHUNK) * CHUNK, _rem)])

        pltpu.make_async_copy(
            src_hbm.at[pl.ds(tid * EDGES_PER_TILE, EDGES_PER_TILE)], sidx,
            gsem0).wait()
        plsc.subcore_barrier()

        gsems = (gsem0, gsem1)
        dsems = (dsem0, dsem1)
        didxs = (didxa, didxb)

        def fetch_didx(cur, b):
            pltpu.async_copy(dst_hbm.at[tid, cur],
                             didxs[b].at[pl.ds(0, 1)], dsems[b])

        def wait_didx(cur, b):
            pltpu.make_async_copy(dst_hbm.at[tid, cur],
                                  didxs[b].at[pl.ds(0, 1)], dsems[b]).wait()

        pltpu.async_copy(h_hbm.at[sidx.at[pl.ds(0, CHUNK)]], rows.at[0],
                         gsem0)
        fetch_didx(0, 0)

        def chunk_pair(kk, carry):
            for p in range(2):
                cur = kk * 2 + p

                @pl.when(cur + 1 < NCHUNK)
                def _prefetch():
                    pltpu.async_copy(
                        h_hbm.at[sidx.at[pl.ds((cur + 1) * CHUNK, CHUNK)]],
                        rows.at[1 - p], gsems[1 - p])
                    fetch_didx(cur + 1, 1 - p)

                pltpu.make_async_copy(
                    h_hbm.at[sidx.at[pl.ds(cur * CHUNK, CHUNK)]],
                    rows.at[p], gsems[p]).wait()
                wait_didx(cur, p)
                pltpu.sync_copy(rows.at[p], acc.at[didxs[p].at[0]], add=True)
            return carry

        assert NCHUNK % 2 == 1
        lax.fori_loop(0, NCHUNK // 2, chunk_pair, 0)
        last = NCHUNK - 1
        pltpu.make_async_copy(
            h_hbm.at[sidx.at[pl.ds(last * CHUNK, CHUNK)]],
            rows.at[0], gsem0).wait()
        wait_didx(last, 0)
        pltpu.sync_copy(rows.at[0], acc.at[didxs[0].at[0]], add=True)

        plsc.subcore_barrier()

        pltpu.sync_copy(
            acc.at[pl.ds(sid * ROWS_PER_TILE, ROWS_PER_TILE)],
            out_hbm.at[cid, pl.ds(sid * ROWS_PER_TILE, ROWS_PER_TILE)])

    return k(h, src2d, dst2d)


_BLK = 400
_GRID = N // _BLK


def _mlp_tc(h, msg, scale, W1, b1, W2, b2, gamma, beta):
    def body(scale_ref, h_ref, m0_ref, m1_ref, W1_ref, b1_ref, W2_ref,
             b2_ref, g_ref, be_ref, o_ref):
        z = scale_ref[0] * h_ref[...] + m0_ref[...] + m1_ref[...]
        a = jnp.dot(z, W1_ref[...], preferred_element_type=jnp.float32)
        a = jnp.maximum(a + b1_ref[...], 0.0)
        zz = jnp.dot(a, W2_ref[...], preferred_element_type=jnp.float32)
        zz = (zz + b2_ref[...]) * g_ref[...] + be_ref[...]
        o_ref[...] = jnp.maximum(zz, 0.0)

    full = lambda s: pl.BlockSpec(s, lambda i: (0,) * len(s))
    return pl.pallas_call(
        body,
        grid=(_GRID,),
        in_specs=[
            pl.BlockSpec(memory_space=pltpu.SMEM),
            pl.BlockSpec((_BLK, D), lambda i: (i, 0)),
            pl.BlockSpec((None, _BLK, D), lambda i: (0, i, 0)),
            pl.BlockSpec((None, _BLK, D), lambda i: (1, i, 0)),
            full((D, 2 * D)),
            full((1, 2 * D)),
            full((2 * D, D)),
            full((1, D)),
            full((1, D)),
            full((1, D)),
        ],
        out_specs=pl.BlockSpec((_BLK, D), lambda i: (i, 0)),
        out_shape=jax.ShapeDtypeStruct((N, D), jnp.float32),
    )(scale, h, msg, msg, W1, b1, W2, b2, gamma, beta)


def _mlp_pool_tc(h, msg, scale, W1, b1, W2, b2, gamma, beta, batch,
                 Wh1, bh1, Wh2, bh2, Wg1, bg1, Wg2, bg2):
    def body(scale_ref, h_ref, m0_ref, m1_ref, W1_ref, b1_ref, W2_ref,
             b2_ref, g_ref, be_ref, batch_ref,
             Wh1_ref, bh1_ref, Wh2_ref, bh2_ref,
             Wg1_ref, bg1_ref, Wg2_ref, bg2_ref,
             o1_ref, o2_ref, P_acc, c_acc):
        i = pl.program_id(0)

        @pl.when(i == 0)
        def _init():
            P_acc[...] = jnp.zeros_like(P_acc)
            c_acc[...] = jnp.zeros_like(c_acc)

        z = scale_ref[0] * h_ref[...] + m0_ref[...] + m1_ref[...]
        a = jnp.dot(z, W1_ref[...], preferred_element_type=jnp.float32)
        a = jnp.maximum(a + b1_ref[...], 0.0)
        zz = jnp.dot(a, W2_ref[...], preferred_element_type=jnp.float32)
        zz = (zz + b2_ref[...]) * g_ref[...] + be_ref[...]

        gids = lax.broadcasted_iota(jnp.int32, (_BLK, G), 1)
        onehot = jnp.where(batch_ref[...] == gids, 1.0, 0.0)
        P_acc[...] += jax.lax.dot_general(
            onehot, zz, (((0,), (0,)), ((), ())),
            preferred_element_type=jnp.float32)
        c_acc[...] += jnp.sum(onehot, axis=0, keepdims=True)

        @pl.when(i == _GRID - 1)
        def _fin():
            cnt = c_acc[...].reshape(G, 1)
            g1 = jnp.dot(P_acc[...], Wh1_ref[...],
                         preferred_element_type=jnp.float32) + cnt * bh1_ref[...]
            g2 = jnp.dot(P_acc[...], Wh2_ref[...],
                         preferred_element_type=jnp.float32) + cnt * bh2_ref[...]
            o1_ref[...] = jnp.dot(g1, Wg1_ref[...],
                                  preferred_element_type=jnp.float32) + bg1_ref[...]
            o2_ref[...] = jnp.dot(g2, Wg2_ref[...],
                                  preferred_element_type=jnp.float32) + bg2_ref[...]

    full = lambda s: pl.BlockSpec(s, lambda i: (0,) * len(s))
    return pl.pallas_call(
        body,
        grid=(_GRID,),
        in_specs=[
            pl.BlockSpec(memory_space=pltpu.SMEM),
            pl.BlockSpec((_BLK, D), lambda i: (i, 0)),
            pl.BlockSpec((None, _BLK, D), lambda i: (0, i, 0)),
            pl.BlockSpec((None, _BLK, D), lambda i: (1, i, 0)),
            full((D, 2 * D)),
            full((1, 2 * D)),
            full((2 * D, D)),
            full((1, D)),
            full((1, D)),
            full((1, D)),
            pl.BlockSpec((_BLK, 1), lambda i: (i, 0)),
            full((D, D)), full((1, D)),
            full((D, D)), full((1, D)),
            full((D, D)), full((1, D)),
            full((D, D)), full((1, D)),
        ],
        out_specs=[full((G, D)), full((G, D))],
        out_shape=[jax.ShapeDtypeStruct((G, D), jnp.float32),
                   jax.ShapeDtypeStruct((G, D), jnp.float32)],
        scratch_shapes=[pltpu.VMEM((G, D), jnp.float32),
                        pltpu.VMEM((1, G), jnp.float32)],
    )(scale, h, msg, msg, W1, b1, W2, b2, gamma, beta, batch,
      Wh1, bh1, Wh2, bh2, Wg1, bg1, Wg2, bg2)


def kernel(x, edge_index, batch, W1, b1, W2, b2, eps, gamma, beta,
           Wh1, bh1, Wh2, bh2, Wg1, bg1, Wg2, bg2):
    nw = NC * NS
    pad_w = EDGES_PER_TILE - REAL_PER_TILE
    src = jnp.pad(edge_index[0].reshape(nw, REAL_PER_TILE),
                  ((0, 0), (0, pad_w))).reshape(nw * EDGES_PER_TILE)
    dst = jnp.pad(edge_index[1].reshape(nw, REAL_PER_TILE),
                  ((0, 0), (0, pad_w)),
                  constant_values=N_PAD - 1).reshape(nw, NCHUNK, 1, CHUNK)
    batch2d = batch.reshape(N, 1)

    h = x
    for l in range(L):
        msg = _segment_sum_sc(h, src, dst)
        scale = (1.0 + eps[l]).reshape(1)
        if l < L - 1:
            h = _mlp_tc(h, msg, scale, W1[l], b1[l].reshape(1, 2 * D),
                        W2[l], b2[l].reshape(1, D),
                        gamma[l].reshape(1, D), beta[l].reshape(1, D))
        else:
            out1, out2 = _mlp_pool_tc(
                h, msg, scale, W1[l], b1[l].reshape(1, 2 * D),
                W2[l], b2[l].reshape(1, D),
                gamma[l].reshape(1, D), beta[l].reshape(1, D), batch2d,
                Wh1, bh1.reshape(1, D), Wh2, bh2.reshape(1, D),
                Wg1, bg1.reshape(1, D), Wg2, bg2.reshape(1, D))
    return (out1, out2)


# speedup vs baseline: 1.9921x; 1.9921x over previous
import functools

import jax
import jax.numpy as jnp
from jax import lax
from jax.experimental import pallas as pl
from jax.experimental.pallas import tpu as pltpu
from jax.experimental.pallas import tpu_sc as plsc

N = 10000
E = 320000
D = 128
L = 5
G = 64

NC = 2
NS = 16
CHUNK = 128
NCHUNK = 79
EDGES_PER_TILE = NCHUNK * CHUNK
REAL_PER_TILE = E // (NC * NS)
ROWS_PER_TILE = 640
N_PAD = NS * ROWS_PER_TILE


def _segment_sum_sc(h, src2d, dst2d):
    mesh = plsc.VectorSubcoreMesh(core_axis_name="c", subcore_axis_name="s")

    @functools.partial(
        pl.kernel,
        out_type=jax.ShapeDtypeStruct((NC, N_PAD, D), jnp.float32),
        mesh=mesh,
        scratch_types=[
            pltpu.VMEM((EDGES_PER_TILE,), jnp.int32),
            pltpu.VMEM((8, CHUNK), jnp.int32),
            pltpu.VMEM((8, CHUNK), jnp.int32),
            pltpu.VMEM((2, CHUNK, D), jnp.float32),
            pltpu.VMEM_SHARED((N_PAD, D), jnp.float32),
            pltpu.SemaphoreType.DMA,
            pltpu.SemaphoreType.DMA,
            pltpu.SemaphoreType.DMA,
            pltpu.SemaphoreType.DMA,
        ],
    )
    def k(h_hbm, src_hbm, dst_hbm, out_hbm, sidx, didxa, didxb, rows, acc,
          gsem0, gsem1, dsem0, dsem1):
        cid = lax.axis_index("c")
        sid = lax.axis_index("s")
        tid = cid * NS + sid

        zeros16 = jnp.zeros((16,), jnp.float32)

        pltpu.async_copy(
            src_hbm.at[pl.ds(tid * EDGES_PER_TILE, EDGES_PER_TILE)], sidx,
            gsem0)

        def zrow(r, carry):
            for j in range(D // 16):
                rows[0, r, pl.ds(j * 16, 16)] = zeros16
            return carry

        lax.fori_loop(0, CHUNK, zrow, 0)
        for t in range(ROWS_PER_TILE // CHUNK):
            pltpu.sync_copy(
                rows.at[0],
                acc.at[pl.ds(sid * ROWS_PER_TILE + t * CHUNK, CHUNK)])
        _rem = ROWS_PER_TILE % CHUNK
        if _rem:
            pltpu.sync_copy(
                rows.at[0, pl.ds(0, _rem)],
                acc.at[pl.ds(sid * ROWS_PER_TILE
                             + (ROWS_PER_TILE // CHUNK) * CHUNK, _rem)])

        pltpu.make_async_copy(
            src_hbm.at[pl.ds(tid * EDGES_PER_TILE, EDGES_PER_TILE)], sidx,
            gsem0).wait()
        plsc.subcore_barrier()

        gsems = (gsem0, gsem1)
        dsems = (dsem0, dsem1)
        didxs = (didxa, didxb)

        def fetch_didx(cur, b):
            pltpu.async_copy(dst_hbm.at[tid, cur],
                             didxs[b].at[pl.ds(0, 1)], dsems[b])

        def wait_didx(cur, b):
            pltpu.make_async_copy(dst_hbm.at[tid, cur],
                                  didxs[b].at[pl.ds(0, 1)], dsems[b]).wait()

        pltpu.async_copy(h_hbm.at[sidx.at[pl.ds(0, CHUNK)]], rows.at[0],
                         gsem0)
        fetch_didx(0, 0)

        def chunk_pair(kk, carry):
            for p in range(2):
                cur = kk * 2 + p

                @pl.when(cur + 1 < NCHUNK)
                def _prefetch():
                    pltpu.async_copy(
                        h_hbm.at[sidx.at[pl.ds((cur + 1) * CHUNK, CHUNK)]],
                        rows.at[1 - p], gsems[1 - p])
                    fetch_didx(cur + 1, 1 - p)

                pltpu.make_async_copy(
                    h_hbm.at[sidx.at[pl.ds(cur * CHUNK, CHUNK)]],
                    rows.at[p], gsems[p]).wait()
                wait_didx(cur, p)
                pltpu.sync_copy(rows.at[p], acc.at[didxs[p].at[0]], add=True)
            return carry

        assert NCHUNK % 2 == 1
        lax.fori_loop(0, NCHUNK // 2, chunk_pair, 0)
        last = NCHUNK - 1
        pltpu.make_async_copy(
            h_hbm.at[sidx.at[pl.ds(last * CHUNK, CHUNK)]],
            rows.at[0], gsem0).wait()
        wait_didx(last, 0)
        pltpu.sync_copy(rows.at[0], acc.at[didxs[0].at[0]], add=True)

        plsc.subcore_barrier()

        pltpu.sync_copy(
            acc.at[pl.ds(sid * ROWS_PER_TILE, ROWS_PER_TILE)],
            out_hbm.at[cid, pl.ds(sid * ROWS_PER_TILE, ROWS_PER_TILE)])

    return k(h, src2d, dst2d)


_BLK = 400
_GRID = N // _BLK


def _mlp_tc(h, msg, scale, W1, b1, W2, b2, gamma, beta):
    def body(scale_ref, h_ref, m0_ref, m1_ref, W1_ref, b1_ref, W2_ref,
             b2_ref, g_ref, be_ref, o_ref):
        z = scale_ref[0] * h_ref[...] + m0_ref[...] + m1_ref[...]
        a = jnp.dot(z, W1_ref[...], preferred_element_type=jnp.float32)
        a = jnp.maximum(a + b1_ref[...], 0.0)
        zz = jnp.dot(a, W2_ref[...], preferred_element_type=jnp.float32)
        zz = (zz + b2_ref[...]) * g_ref[...] + be_ref[...]
        o_ref[...] = jnp.maximum(zz, 0.0)

    full = lambda s: pl.BlockSpec(s, lambda i: (0,) * len(s))
    return pl.pallas_call(
        body,
        grid=(_GRID,),
        in_specs=[
            pl.BlockSpec(memory_space=pltpu.SMEM),
            pl.BlockSpec((_BLK, D), lambda i: (i, 0)),
            pl.BlockSpec((None, _BLK, D), lambda i: (0, i, 0)),
            pl.BlockSpec((None, _BLK, D), lambda i: (1, i, 0)),
            full((D, 2 * D)),
            full((1, 2 * D)),
            full((2 * D, D)),
            full((1, D)),
            full((1, D)),
            full((1, D)),
        ],
        out_specs=pl.BlockSpec((_BLK, D), lambda i: (i, 0)),
        out_shape=jax.ShapeDtypeStruct((N, D), jnp.float32),
    )(scale, h, msg, msg, W1, b1, W2, b2, gamma, beta)


def _mlp_pool_tc(h, msg, scale, W1, b1, W2, b2, gamma, beta, batch,
                 Wh1, bh1, Wh2, bh2, Wg1, bg1, Wg2, bg2):
    def body(scale_ref, h_ref, m0_ref, m1_ref, W1_ref, b1_ref, W2_ref,
             b2_ref, g_ref, be_ref, batch_ref,
             Wh1_ref, bh1_ref, Wh2_ref, bh2_ref,
             Wg1_ref, bg1_ref, Wg2_ref, bg2_ref,
             o1_ref, o2_ref, P_acc, c_acc):
        i = pl.program_id(0)

        @pl.when(i == 0)
        def _init():
            P_acc[...] = jnp.zeros_like(P_acc)
            c_acc[...] = jnp.zeros_like(c_acc)

        z = scale_ref[0] * h_ref[...] + m0_ref[...] + m1_ref[...]
        a = jnp.dot(z, W1_ref[...], preferred_element_type=jnp.float32)
        a = jnp.maximum(a + b1_ref[...], 0.0)
        zz = jnp.dot(a, W2_ref[...], preferred_element_type=jnp.float32)
        zz = (zz + b2_ref[...]) * g_ref[...] + be_ref[...]

        gids = lax.broadcasted_iota(jnp.int32, (_BLK, G), 1)
        onehot = jnp.where(batch_ref[...] == gids, 1.0, 0.0)
        P_acc[...] += jax.lax.dot_general(
            onehot, zz, (((0,), (0,)), ((), ())),
            preferred_element_type=jnp.float32)
        c_acc[...] += jnp.sum(onehot, axis=0, keepdims=True)

        @pl.when(i == _GRID - 1)
        def _fin():
            cnt = c_acc[...].reshape(G, 1)
            g1 = jnp.dot(P_acc[...], Wh1_ref[...],
                         preferred_element_type=jnp.float32) + cnt * bh1_ref[...]
            g2 = jnp.dot(P_acc[...], Wh2_ref[...],
                         preferred_element_type=jnp.float32) + cnt * bh2_ref[...]
            o1_ref[...] = jnp.dot(g1, Wg1_ref[...],
                                  preferred_element_type=jnp.float32) + bg1_ref[...]
            o2_ref[...] = jnp.dot(g2, Wg2_ref[...],
                                  preferred_element_type=jnp.float32) + bg2_ref[...]

    full = lambda s: pl.BlockSpec(s, lambda i: (0,) * len(s))
    return pl.pallas_call(
        body,
        grid=(_GRID,),
        in_specs=[
            pl.BlockSpec(memory_space=pltpu.SMEM),
            pl.BlockSpec((_BLK, D), lambda i: (i, 0)),
            pl.BlockSpec((None, _BLK, D), lambda i: (0, i, 0)),
            pl.BlockSpec((None, _BLK, D), lambda i: (1, i, 0)),
            full((D, 2 * D)),
            full((1, 2 * D)),
            full((2 * D, D)),
            full((1, D)),
            full((1, D)),
            full((1, D)),
            pl.BlockSpec((_BLK, 1), lambda i: (i, 0)),
            full((D, D)), full((1, D)),
            full((D, D)), full((1, D)),
            full((D, D)), full((1, D)),
            full((D, D)), full((1, D)),
        ],
        out_specs=[full((G, D)), full((G, D))],
        out_shape=[jax.ShapeDtypeStruct((G, D), jnp.float32),
                   jax.ShapeDtypeStruct((G, D), jnp.float32)],
        scratch_shapes=[pltpu.VMEM((G, D), jnp.float32),
                        pltpu.VMEM((1, G), jnp.float32)],
    )(scale, h, msg, msg, W1, b1, W2, b2, gamma, beta, batch,
      Wh1, bh1, Wh2, bh2, Wg1, bg1, Wg2, bg2)


def kernel(x, edge_index, batch, W1, b1, W2, b2, eps, gamma, beta,
           Wh1, bh1, Wh2, bh2, Wg1, bg1, Wg2, bg2):
    nw = NC * NS
    pad_w = EDGES_PER_TILE - REAL_PER_TILE
    spare = N_PAD - N
    pad_dst = N + (jnp.arange(pad_w, dtype=jnp.int32) % spare)
    pad_src = jnp.arange(pad_w, dtype=jnp.int32) * 89 % N
    src = jnp.concatenate(
        [edge_index[0].reshape(nw, REAL_PER_TILE),
         jnp.broadcast_to(pad_src, (nw, pad_w))],
        axis=1).reshape(nw * EDGES_PER_TILE)
    dst = jnp.concatenate(
        [edge_index[1].reshape(nw, REAL_PER_TILE),
         jnp.broadcast_to(pad_dst, (nw, pad_w))],
        axis=1).reshape(nw, NCHUNK, 1, CHUNK)
    batch2d = batch.reshape(N, 1)

    h = x
    for l in range(L):
        msg = _segment_sum_sc(h, src, dst)
        scale = (1.0 + eps[l]).reshape(1)
        if l < L - 1:
            h = _mlp_tc(h, msg, scale, W1[l], b1[l].reshape(1, 2 * D),
                        W2[l], b2[l].reshape(1, D),
                        gamma[l].reshape(1, D), beta[l].reshape(1, D))
        else:
            out1, out2 = _mlp_pool_tc(
                h, msg, scale, W1[l], b1[l].reshape(1, 2 * D),
                W2[l], b2[l].reshape(1, D),
                gamma[l].reshape(1, D), beta[l].reshape(1, D), batch2d,
                Wh1, bh1.reshape(1, D), Wh2, bh2.reshape(1, D),
                Wg1, bg1.reshape(1, D), Wg2, bg2.reshape(1, D))
    return (out1, out2)


# speedup vs baseline: 2.1328x; 1.0706x over previous
import functools

import jax
import jax.numpy as jnp
from jax import lax
from jax.experimental import pallas as pl
from jax.experimental.pallas import tpu as pltpu
from jax.experimental.pallas import tpu_sc as plsc

N = 10000
E = 320000
D = 128
L = 5
G = 64

NC = 2
NS = 16
CHUNK = 128
NCHUNK = 79
EDGES_PER_TILE = NCHUNK * CHUNK
REAL_PER_TILE = E // (NC * NS)
ROWS_PER_TILE = 640
N_PAD = NS * ROWS_PER_TILE


def _segment_sum_sc(h, src2d, dst2d):
    mesh = plsc.VectorSubcoreMesh(core_axis_name="c", subcore_axis_name="s")

    @functools.partial(
        pl.kernel,
        out_type=jax.ShapeDtypeStruct((NC, N_PAD, D), jnp.float32),
        mesh=mesh,
        scratch_types=[
            pltpu.VMEM((EDGES_PER_TILE,), jnp.int32),
            pltpu.VMEM((8, CHUNK), jnp.int32),
            pltpu.VMEM((8, CHUNK), jnp.int32),
            pltpu.VMEM((2, CHUNK, D), jnp.float32),
            pltpu.VMEM_SHARED((N_PAD, D), jnp.float32),
            pltpu.SemaphoreType.DMA,
            pltpu.SemaphoreType.DMA,
            pltpu.SemaphoreType.DMA,
            pltpu.SemaphoreType.DMA,
        ],
    )
    def k(h_hbm, src_hbm, dst_hbm, out_hbm, sidx, didxa, didxb, rows, acc,
          gsem0, gsem1, dsem0, dsem1):
        cid = lax.axis_index("c")
        sid = lax.axis_index("s")
        tid = cid * NS + sid

        zeros16 = jnp.zeros((16,), jnp.float32)

        pltpu.async_copy(
            src_hbm.at[pl.ds(tid * EDGES_PER_TILE, EDGES_PER_TILE)], sidx,
            gsem0)

        def zrow(r, carry):
            for j in range(D // 16):
                rows[0, r, pl.ds(j * 16, 16)] = zeros16
            return carry

        lax.fori_loop(0, CHUNK, zrow, 0)
        for t in range(ROWS_PER_TILE // CHUNK):
            pltpu.sync_copy(
                rows.at[0],
                acc.at[pl.ds(sid * ROWS_PER_TILE + t * CHUNK, CHUNK)])
        _rem = ROWS_PER_TILE % CHUNK
        if _rem:
            pltpu.sync_copy(
                rows.at[0, pl.ds(0, _rem)],
                acc.at[pl.ds(sid * ROWS_PER_TILE
                             + (ROWS_PER_TILE // CHUNK) * CHUNK, _rem)])

        pltpu.make_async_copy(
            src_hbm.at[pl.ds(tid * EDGES_PER_TILE, EDGES_PER_TILE)], sidx,
            gsem0).wait()
        plsc.subcore_barrier()

        gsems = (gsem0, gsem1)
        dsems = (dsem0, dsem1)
        didxs = (didxa, didxb)

        def fetch_didx(cur, b):
            pltpu.async_copy(dst_hbm.at[tid, cur],
                             didxs[b].at[pl.ds(0, 1)], dsems[b])

        def wait_didx(cur, b):
            pltpu.make_async_copy(dst_hbm.at[tid, cur],
                                  didxs[b].at[pl.ds(0, 1)], dsems[b]).wait()

        pltpu.async_copy(h_hbm.at[sidx.at[pl.ds(0, CHUNK)]], rows.at[0],
                         gsem0)
        fetch_didx(0, 0)

        def chunk_pair(kk, carry):
            for p in range(2):
                cur = kk * 2 + p

                @pl.when(cur + 1 < NCHUNK)
                def _prefetch():
                    pltpu.async_copy(
                        h_hbm.at[sidx.at[pl.ds((cur + 1) * CHUNK, CHUNK)]],
                        rows.at[1 - p], gsems[1 - p])
                    fetch_didx(cur + 1, 1 - p)

                pltpu.make_async_copy(
                    h_hbm.at[sidx.at[pl.ds(cur * CHUNK, CHUNK)]],
                    rows.at[p], gsems[p]).wait()
                wait_didx(cur, p)
                pltpu.sync_copy(rows.at[p], acc.at[didxs[p].at[0]], add=True)
            return carry

        assert NCHUNK % 2 == 1
        lax.fori_loop(0, NCHUNK // 2, chunk_pair, 0)
        last = NCHUNK - 1
        pltpu.make_async_copy(
            h_hbm.at[sidx.at[pl.ds(last * CHUNK, CHUNK)]],
            rows.at[0], gsem0).wait()
        wait_didx(last, 0)
        pltpu.sync_copy(rows.at[0], acc.at[didxs[0].at[0]], add=True)

        plsc.subcore_barrier()

        pltpu.sync_copy(
            acc.at[pl.ds(sid * ROWS_PER_TILE, ROWS_PER_TILE)],
            out_hbm.at[cid, pl.ds(sid * ROWS_PER_TILE, ROWS_PER_TILE)])

    return k(h, src2d, dst2d)


_BLK = 1000
_GRID = N // _BLK


def _mlp_tc(h, msg, scale, W1, b1, W2, b2, gamma, beta):
    def body(scale_ref, h_ref, m0_ref, m1_ref, W1_ref, b1_ref, W2_ref,
             b2_ref, g_ref, be_ref, o_ref):
        z = scale_ref[0] * h_ref[...] + m0_ref[...] + m1_ref[...]
        a = jnp.dot(z, W1_ref[...], preferred_element_type=jnp.float32)
        a = jnp.maximum(a + b1_ref[...], 0.0)
        zz = jnp.dot(a, W2_ref[...], preferred_element_type=jnp.float32)
        zz = (zz + b2_ref[...]) * g_ref[...] + be_ref[...]
        o_ref[...] = jnp.maximum(zz, 0.0)

    full = lambda s: pl.BlockSpec(s, lambda i: (0,) * len(s))
    return pl.pallas_call(
        body,
        grid=(_GRID,),
        in_specs=[
            pl.BlockSpec(memory_space=pltpu.SMEM),
            pl.BlockSpec((_BLK, D), lambda i: (i, 0)),
            pl.BlockSpec((None, _BLK, D), lambda i: (0, i, 0)),
            pl.BlockSpec((None, _BLK, D), lambda i: (1, i, 0)),
            full((D, 2 * D)),
            full((1, 2 * D)),
            full((2 * D, D)),
            full((1, D)),
            full((1, D)),
            full((1, D)),
        ],
        out_specs=pl.BlockSpec((_BLK, D), lambda i: (i, 0)),
        out_shape=jax.ShapeDtypeStruct((N, D), jnp.float32),
    )(scale, h, msg, msg, W1, b1, W2, b2, gamma, beta)


def _mlp_pool_tc(h, msg, scale, W1, b1, W2, b2, gamma, beta, batch,
                 Wh1, bh1, Wh2, bh2, Wg1, bg1, Wg2, bg2):
    def body(scale_ref, h_ref, m0_ref, m1_ref, W1_ref, b1_ref, W2_ref,
             b2_ref, g_ref, be_ref, batch_ref,
             Wh1_ref, bh1_ref, Wh2_ref, bh2_ref,
             Wg1_ref, bg1_ref, Wg2_ref, bg2_ref,
             o1_ref, o2_ref, P_acc, c_acc):
        i = pl.program_id(0)

        @pl.when(i == 0)
        def _init():
            P_acc[...] = jnp.zeros_like(P_acc)
            c_acc[...] = jnp.zeros_like(c_acc)

        z = scale_ref[0] * h_ref[...] + m0_ref[...] + m1_ref[...]
        a = jnp.dot(z, W1_ref[...], preferred_element_type=jnp.float32)
        a = jnp.maximum(a + b1_ref[...], 0.0)
        zz = jnp.dot(a, W2_ref[...], preferred_element_type=jnp.float32)
        zz = (zz + b2_ref[...]) * g_ref[...] + be_ref[...]

        gids = lax.broadcasted_iota(jnp.int32, (_BLK, G), 1)
        onehot = jnp.where(batch_ref[...] == gids, 1.0, 0.0)
        P_acc[...] += jax.lax.dot_general(
            onehot, zz, (((0,), (0,)), ((), ())),
            preferred_element_type=jnp.float32)
        c_acc[...] += jnp.sum(onehot, axis=0, keepdims=True)

        @pl.when(i == _GRID - 1)
        def _fin():
            cnt = c_acc[...].reshape(G, 1)
            g1 = jnp.dot(P_acc[...], Wh1_ref[...],
                         preferred_element_type=jnp.float32) + cnt * bh1_ref[...]
            g2 = jnp.dot(P_acc[...], Wh2_ref[...],
                         preferred_element_type=jnp.float32) + cnt * bh2_ref[...]
            o1_ref[...] = jnp.dot(g1, Wg1_ref[...],
                                  preferred_element_type=jnp.float32) + bg1_ref[...]
            o2_ref[...] = jnp.dot(g2, Wg2_ref[...],
                                  preferred_element_type=jnp.float32) + bg2_ref[...]

    full = lambda s: pl.BlockSpec(s, lambda i: (0,) * len(s))
    return pl.pallas_call(
        body,
        grid=(_GRID,),
        in_specs=[
            pl.BlockSpec(memory_space=pltpu.SMEM),
            pl.BlockSpec((_BLK, D), lambda i: (i, 0)),
            pl.BlockSpec((None, _BLK, D), lambda i: (0, i, 0)),
            pl.BlockSpec((None, _BLK, D), lambda i: (1, i, 0)),
            full((D, 2 * D)),
            full((1, 2 * D)),
            full((2 * D, D)),
            full((1, D)),
            full((1, D)),
            full((1, D)),
            pl.BlockSpec((_BLK, 1), lambda i: (i, 0)),
            full((D, D)), full((1, D)),
            full((D, D)), full((1, D)),
            full((D, D)), full((1, D)),
            full((D, D)), full((1, D)),
        ],
        out_specs=[full((G, D)), full((G, D))],
        out_shape=[jax.ShapeDtypeStruct((G, D), jnp.float32),
                   jax.ShapeDtypeStruct((G, D), jnp.float32)],
        scratch_shapes=[pltpu.VMEM((G, D), jnp.float32),
                        pltpu.VMEM((1, G), jnp.float32)],
    )(scale, h, msg, msg, W1, b1, W2, b2, gamma, beta, batch,
      Wh1, bh1, Wh2, bh2, Wg1, bg1, Wg2, bg2)


def kernel(x, edge_index, batch, W1, b1, W2, b2, eps, gamma, beta,
           Wh1, bh1, Wh2, bh2, Wg1, bg1, Wg2, bg2):
    nw = NC * NS
    pad_w = EDGES_PER_TILE - REAL_PER_TILE
    spare = N_PAD - N
    pad_dst = N + (jnp.arange(pad_w, dtype=jnp.int32) % spare)
    pad_src = jnp.arange(pad_w, dtype=jnp.int32) * 89 % N
    src = jnp.concatenate(
        [edge_index[0].reshape(nw, REAL_PER_TILE),
         jnp.broadcast_to(pad_src, (nw, pad_w))],
        axis=1).reshape(nw * EDGES_PER_TILE)
    dst = jnp.concatenate(
        [edge_index[1].reshape(nw, REAL_PER_TILE),
         jnp.broadcast_to(pad_dst, (nw, pad_w))],
        axis=1).reshape(nw, NCHUNK, 1, CHUNK)
    batch2d = batch.reshape(N, 1)

    h = x
    for l in range(L):
        msg = _segment_sum_sc(h, src, dst)
        scale = (1.0 + eps[l]).reshape(1)
        if l < L - 1:
            h = _mlp_tc(h, msg, scale, W1[l], b1[l].reshape(1, 2 * D),
                        W2[l], b2[l].reshape(1, D),
                        gamma[l].reshape(1, D), beta[l].reshape(1, D))
        else:
            out1, out2 = _mlp_pool_tc(
                h, msg, scale, W1[l], b1[l].reshape(1, 2 * D),
                W2[l], b2[l].reshape(1, D),
                gamma[l].reshape(1, D), beta[l].reshape(1, D), batch2d,
                Wh1, bh1.reshape(1, D), Wh2, bh2.reshape(1, D),
                Wg1, bg1.reshape(1, D), Wg2, bg2.reshape(1, D))
    return (out1, out2)


# speedup vs baseline: 2.1721x; 1.0185x over previous
import functools

import jax
import jax.numpy as jnp
from jax import lax
from jax.experimental import pallas as pl
from jax.experimental.pallas import tpu as pltpu
from jax.experimental.pallas import tpu_sc as plsc

N = 10000
E = 320000
D = 128
L = 5
G = 64

NC = 2
NS = 16
CHUNK = 128
NCHUNK = 79
EDGES_PER_TILE = NCHUNK * CHUNK
REAL_PER_TILE = E // (NC * NS)
ROWS_PER_TILE = 640
N_PAD = NS * ROWS_PER_TILE


def _segment_sum_sc(h, src2d, dst2d):
    mesh = plsc.VectorSubcoreMesh(core_axis_name="c", subcore_axis_name="s")

    @functools.partial(
        pl.kernel,
        out_type=jax.ShapeDtypeStruct((NC, N_PAD, D), jnp.float32),
        mesh=mesh,
        scratch_types=[
            pltpu.VMEM((EDGES_PER_TILE,), jnp.int32),
            pltpu.VMEM((8, CHUNK), jnp.int32),
            pltpu.VMEM((8, CHUNK), jnp.int32),
            pltpu.VMEM((2, CHUNK, D), jnp.float32),
            pltpu.VMEM_SHARED((N_PAD, D), jnp.float32),
            pltpu.SemaphoreType.DMA,
            pltpu.SemaphoreType.DMA,
            pltpu.SemaphoreType.DMA,
            pltpu.SemaphoreType.DMA,
        ],
    )
    def k(h_hbm, src_hbm, dst_hbm, out_hbm, sidx, didxa, didxb, rows, acc,
          gsem0, gsem1, dsem0, dsem1):
        cid = lax.axis_index("c")
        sid = lax.axis_index("s")
        tid = cid * NS + sid

        zeros16 = jnp.zeros((16,), jnp.float32)

        pltpu.async_copy(
            src_hbm.at[pl.ds(tid * EDGES_PER_TILE, EDGES_PER_TILE)], sidx,
            gsem0)

        def zrow(r, carry):
            for j in range(D // 16):
                rows[0, r, pl.ds(j * 16, 16)] = zeros16
            return carry

        lax.fori_loop(0, CHUNK, zrow, 0)
        for t in range(ROWS_PER_TILE // CHUNK):
            pltpu.sync_copy(
                rows.at[0],
                acc.at[pl.ds(sid * ROWS_PER_TILE + t * CHUNK, CHUNK)])
        _rem = ROWS_PER_TILE % CHUNK
        if _rem:
            pltpu.sync_copy(
                rows.at[0, pl.ds(0, _rem)],
                acc.at[pl.ds(sid * ROWS_PER_TILE
                             + (ROWS_PER_TILE // CHUNK) * CHUNK, _rem)])

        pltpu.make_async_copy(
            src_hbm.at[pl.ds(tid * EDGES_PER_TILE, EDGES_PER_TILE)], sidx,
            gsem0).wait()
        plsc.subcore_barrier()

        gsems = (gsem0, gsem1)
        dsems = (dsem0, dsem1)
        didxs = (didxa, didxb)

        def fetch_didx(cur, b):
            pltpu.async_copy(dst_hbm.at[tid, cur],
                             didxs[b].at[pl.ds(0, 1)], dsems[b])

        def wait_didx(cur, b):
            pltpu.make_async_copy(dst_hbm.at[tid, cur],
                                  didxs[b].at[pl.ds(0, 1)], dsems[b]).wait()

        pltpu.async_copy(h_hbm.at[sidx.at[pl.ds(0, CHUNK)]], rows.at[0],
                         gsem0)
        fetch_didx(0, 0)

        def chunk_pair(kk, carry):
            for p in range(2):
                cur = kk * 2 + p

                @pl.when(cur + 1 < NCHUNK)
                def _prefetch():
                    pltpu.async_copy(
                        h_hbm.at[sidx.at[pl.ds((cur + 1) * CHUNK, CHUNK)]],
                        rows.at[1 - p], gsems[1 - p])
                    fetch_didx(cur + 1, 1 - p)

                pltpu.make_async_copy(
                    h_hbm.at[sidx.at[pl.ds(cur * CHUNK, CHUNK)]],
                    rows.at[p], gsems[p]).wait()
                wait_didx(cur, p)
                pltpu.sync_copy(rows.at[p], acc.at[didxs[p].at[0]], add=True)
            return carry

        assert NCHUNK % 2 == 1
        lax.fori_loop(0, NCHUNK // 2, chunk_pair, 0)
        last = NCHUNK - 1
        pltpu.make_async_copy(
            h_hbm.at[sidx.at[pl.ds(last * CHUNK, CHUNK)]],
            rows.at[0], gsem0).wait()
        wait_didx(last, 0)
        pltpu.sync_copy(rows.at[0], acc.at[didxs[0].at[0]], add=True)

        plsc.subcore_barrier()

        pltpu.sync_copy(
            acc.at[pl.ds(sid * ROWS_PER_TILE, ROWS_PER_TILE)],
            out_hbm.at[cid, pl.ds(sid * ROWS_PER_TILE, ROWS_PER_TILE)])

    return k(h, src2d, dst2d)


_BLK = 2000
_GRID = N // _BLK


def _mlp_tc(h, msg, scale, W1, b1, W2, b2, gamma, beta):
    def body(scale_ref, h_ref, m0_ref, m1_ref, W1_ref, b1_ref, W2_ref,
             b2_ref, g_ref, be_ref, o_ref):
        z = scale_ref[0] * h_ref[...] + m0_ref[...] + m1_ref[...]
        a = jnp.dot(z, W1_ref[...], preferred_element_type=jnp.float32)
        a = jnp.maximum(a + b1_ref[...], 0.0)
        zz = jnp.dot(a, W2_ref[...], preferred_element_type=jnp.float32)
        zz = (zz + b2_ref[...]) * g_ref[...] + be_ref[...]
        o_ref[...] = jnp.maximum(zz, 0.0)

    full = lambda s: pl.BlockSpec(s, lambda i: (0,) * len(s))
    return pl.pallas_call(
        body,
        grid=(_GRID,),
        in_specs=[
            pl.BlockSpec(memory_space=pltpu.SMEM),
            pl.BlockSpec((_BLK, D), lambda i: (i, 0)),
            pl.BlockSpec((None, _BLK, D), lambda i: (0, i, 0)),
            pl.BlockSpec((None, _BLK, D), lambda i: (1, i, 0)),
            full((D, 2 * D)),
            full((1, 2 * D)),
            full((2 * D, D)),
            full((1, D)),
            full((1, D)),
            full((1, D)),
        ],
        out_specs=pl.BlockSpec((_BLK, D), lambda i: (i, 0)),
        out_shape=jax.ShapeDtypeStruct((N, D), jnp.float32),
    )(scale, h, msg, msg, W1, b1, W2, b2, gamma, beta)


def _mlp_pool_tc(h, msg, scale, W1, b1, W2, b2, gamma, beta, batch,
                 Wh1, bh1, Wh2, bh2, Wg1, bg1, Wg2, bg2):
    def body(scale_ref, h_ref, m0_ref, m1_ref, W1_ref, b1_ref, W2_ref,
             b2_ref, g_ref, be_ref, batch_ref,
             Wh1_ref, bh1_ref, Wh2_ref, bh2_ref,
             Wg1_ref, bg1_ref, Wg2_ref, bg2_ref,
             o1_ref, o2_ref, P_acc, c_acc):
        i = pl.program_id(0)

        @pl.when(i == 0)
        def _init():
            P_acc[...] = jnp.zeros_like(P_acc)
            c_acc[...] = jnp.zeros_like(c_acc)

        z = scale_ref[0] * h_ref[...] + m0_ref[...] + m1_ref[...]
        a = jnp.dot(z, W1_ref[...], preferred_element_type=jnp.float32)
        a = jnp.maximum(a + b1_ref[...], 0.0)
        zz = jnp.dot(a, W2_ref[...], preferred_element_type=jnp.float32)
        zz = (zz + b2_ref[...]) * g_ref[...] + be_ref[...]

        gids = lax.broadcasted_iota(jnp.int32, (_BLK, G), 1)
        onehot = jnp.where(batch_ref[...] == gids, 1.0, 0.0)
        P_acc[...] += jax.lax.dot_general(
            onehot, zz, (((0,), (0,)), ((), ())),
            preferred_element_type=jnp.float32)
        c_acc[...] += jnp.sum(onehot, axis=0, keepdims=True)

        @pl.when(i == _GRID - 1)
        def _fin():
            cnt = c_acc[...].reshape(G, 1)
            g1 = jnp.dot(P_acc[...], Wh1_ref[...],
                         preferred_element_type=jnp.float32) + cnt * bh1_ref[...]
            g2 = jnp.dot(P_acc[...], Wh2_ref[...],
                         preferred_element_type=jnp.float32) + cnt * bh2_ref[...]
            o1_ref[...] = jnp.dot(g1, Wg1_ref[...],
                                  preferred_element_type=jnp.float32) + bg1_ref[...]
            o2_ref[...] = jnp.dot(g2, Wg2_ref[...],
                                  preferred_element_type=jnp.float32) + bg2_ref[...]

    full = lambda s: pl.BlockSpec(s, lambda i: (0,) * len(s))
    return pl.pallas_call(
        body,
        grid=(_GRID,),
        in_specs=[
            pl.BlockSpec(memory_space=pltpu.SMEM),
            pl.BlockSpec((_BLK, D), lambda i: (i, 0)),
            pl.BlockSpec((None, _BLK, D), lambda i: (0, i, 0)),
            pl.BlockSpec((None, _BLK, D), lambda i: (1, i, 0)),
            full((D, 2 * D)),
            full((1, 2 * D)),
            full((2 * D, D)),
            full((1, D)),
            full((1, D)),
            full((1, D)),
            pl.BlockSpec((_BLK, 1), lambda i: (i, 0)),
            full((D, D)), full((1, D)),
            full((D, D)), full((1, D)),
            full((D, D)), full((1, D)),
            full((D, D)), full((1, D)),
        ],
        out_specs=[full((G, D)), full((G, D))],
        out_shape=[jax.ShapeDtypeStruct((G, D), jnp.float32),
                   jax.ShapeDtypeStruct((G, D), jnp.float32)],
        scratch_shapes=[pltpu.VMEM((G, D), jnp.float32),
                        pltpu.VMEM((1, G), jnp.float32)],
    )(scale, h, msg, msg, W1, b1, W2, b2, gamma, beta, batch,
      Wh1, bh1, Wh2, bh2, Wg1, bg1, Wg2, bg2)


def kernel(x, edge_index, batch, W1, b1, W2, b2, eps, gamma, beta,
           Wh1, bh1, Wh2, bh2, Wg1, bg1, Wg2, bg2):
    nw = NC * NS
    pad_w = EDGES_PER_TILE - REAL_PER_TILE
    spare = N_PAD - N
    pad_dst = N + (jnp.arange(pad_w, dtype=jnp.int32) % spare)
    pad_src = jnp.arange(pad_w, dtype=jnp.int32) * 89 % N
    src = jnp.concatenate(
        [edge_index[0].reshape(nw, REAL_PER_TILE),
         jnp.broadcast_to(pad_src, (nw, pad_w))],
        axis=1).reshape(nw * EDGES_PER_TILE)
    dst = jnp.concatenate(
        [edge_index[1].reshape(nw, REAL_PER_TILE),
         jnp.broadcast_to(pad_dst, (nw, pad_w))],
        axis=1).reshape(nw, NCHUNK, 1, CHUNK)
    batch2d = batch.reshape(N, 1)

    h = x
    for l in range(L):
        msg = _segment_sum_sc(h, src, dst)
        scale = (1.0 + eps[l]).reshape(1)
        if l < L - 1:
            h = _mlp_tc(h, msg, scale, W1[l], b1[l].reshape(1, 2 * D),
                        W2[l], b2[l].reshape(1, D),
                        gamma[l].reshape(1, D), beta[l].reshape(1, D))
        else:
            out1, out2 = _mlp_pool_tc(
                h, msg, scale, W1[l], b1[l].reshape(1, 2 * D),
                W2[l], b2[l].reshape(1, D),
                gamma[l].reshape(1, D), beta[l].reshape(1, D), batch2d,
                Wh1, bh1.reshape(1, D), Wh2, bh2.reshape(1, D),
                Wg1, bg1.reshape(1, D), Wg2, bg2.reshape(1, D))
    return (out1, out2)
